# Initial kernel scaffold; baseline (speedup 1.0000x reference)
import jax
import jax.numpy as jnp
from jax.experimental import pallas as pl


def kernel(adjacency_matrix, timestep, arrivals, departures, is_hard_to_match, active_agents, emb_W1, emb_b1, emb_W2, emb_b2, gat_W, gat_a_src, gat_a_dst, gat_b, sel_W, sel_b):
    raise NotImplementedError("write your pallas kernel here")


# speedup vs baseline: 5059.5378x; 5059.5378x over previous
import jax
import jax.numpy as jnp
from jax.experimental import pallas as pl

N = 1024
HID = 128
LAYERS = 3
_F32 = jnp.float32


def _leaky(x):
    return jnp.where(x >= 0, x, 0.2 * x)


def _body(adjT_ref, ts_ref, arr_ref, dep_ref, ihm_ref, act_col_ref, act_row_ref,
          emb_W1_ref, emb_b1_ref, emb_W2_ref, emb_b2_ref,
          gat_W_ref, gat_a_src_ref, gat_a_dst_ref, gat_b_ref,
          sel_W_ref, sel_b_ref, out_ref):
    f32 = _F32
    ts = ts_ref[0, 0]
    arr = arr_ref[:]
    dep = dep_ref[:]
    ihm = ihm_ref[:]
    act_raw_col = act_col_ref[:]
    act_col = (act_raw_col > 0).astype(f32)
    act_row_ok = act_row_ref[:] > 0

    progress = (ts - arr) / (dep - arr)
    w1p = emb_W1_ref[0:1, :]
    w1h = emb_W1_ref[1:2, :]
    x0 = progress * w1p + ihm * w1h + emb_b1_ref[:]
    x = jax.lax.dot_general(x0, emb_W2_ref[:], (((1,), (0,)), ((), ())),
                            preferred_element_type=f32) + emb_b2_ref[:]

    mask = (adjT_ref[:] > 0) & act_row_ok & (act_raw_col > 0)

    neg_inf = jnp.array(-jnp.inf, f32)
    h = x
    for l in range(LAYERS):
        W = gat_W_ref[l]
        a_s = gat_a_src_ref[l:l + 1, :]
        a_d = gat_a_dst_ref[l:l + 1, :]
        b = gat_b_ref[l:l + 1, :]

        h1 = jax.lax.dot_general(h, W, (((1,), (0,)), ((), ())),
                                 preferred_element_type=f32)
        asrc_col = jax.lax.dot_general(h1, a_s, (((1,), (1,)), ((), ())),
                                       preferred_element_type=f32)
        adst_col = jax.lax.dot_general(h1, a_d, (((1,), (1,)), ((), ())),
                                       preferred_element_type=f32)
        asrc_row = jax.lax.dot_general(a_s, h1, (((1,), (1,)), ((), ())),
                                       preferred_element_type=f32)

        eT = _leaky(adst_col + asrc_row)
        e_self = _leaky(adst_col + asrc_col)
        m = jnp.max(jnp.where(mask, eT, neg_inf), axis=1, keepdims=True)
        m = jnp.maximum(m, e_self)
        exT = jnp.where(mask, jnp.exp(eT - m), 0.0)
        ex_self = jnp.exp(e_self - m)
        denom = jnp.sum(exT, axis=1, keepdims=True) + ex_self
        inv = 1.0 / (denom + 1e-16)

        agg = jax.lax.dot_general(exT, h1, (((1,), (0,)), ((), ())),
                                  preferred_element_type=f32)
        out = agg * inv + (ex_self * inv) * h1 + b
        h = jnp.maximum(out, 0.0) if l < LAYERS - 1 else out

    x = x + h
    mu = jnp.mean(x, axis=1, keepdims=True)
    xc = x - mu
    var = jnp.mean(xc * xc, axis=1, keepdims=True)
    xn = xc * jax.lax.rsqrt(var + 1e-5)
    logit = jax.lax.dot_general(xn, sel_W_ref[:], (((1,), (0,)), ((), ())),
                                preferred_element_type=f32) + sel_b_ref[0, 0]
    y = jax.nn.sigmoid(logit) * act_col
    any_active = jnp.sum(act_raw_col) != 0.0
    out_ref[:] = jnp.where(any_active, y, jnp.zeros_like(y))


def kernel(adjacency_matrix, timestep, arrivals, departures, is_hard_to_match,
           active_agents, emb_W1, emb_b1, emb_W2, emb_b2, gat_W, gat_a_src,
           gat_a_dst, gat_b, sel_W, sel_b):
    f32 = _F32
    adjT = jnp.transpose(adjacency_matrix)
    ts = jnp.full((1, 1), timestep, f32)
    col = lambda v: v.astype(f32).reshape(N, 1)
    row = lambda v: v.astype(f32).reshape(1, -1)
    args = (
        adjT, ts, col(arrivals), col(departures), col(is_hard_to_match),
        col(active_agents), row(active_agents),
        emb_W1, row(emb_b1), emb_W2, row(emb_b2),
        gat_W, gat_a_src, gat_a_dst, gat_b,
        sel_W, row(sel_b),
    )
    out = pl.pallas_call(
        _body,
        out_shape=jax.ShapeDtypeStruct((N, 1), f32),
    )(*args)
    return out


# speedup vs baseline: 5723.5013x; 1.1312x over previous
import jax
import jax.numpy as jnp
from jax.experimental import pallas as pl

N = 1024
HID = 128
LAYERS = 3
_F32 = jnp.float32


def _leaky(x):
    return jnp.where(x >= 0, x, 0.2 * x)


def _body(adj_ref, ts_ref, arr_ref, dep_ref, ihm_ref, act_col_ref, act_row_ref,
          emb_W1_ref, emb_b1_ref, emb_W2_ref, emb_b2_ref,
          gat_W_ref, gat_a_src_ref, gat_a_dst_ref, gat_b_ref,
          sel_W_ref, sel_b_ref, out_ref):
    f32 = _F32
    ts = ts_ref[0, 0]
    arr = arr_ref[:]
    dep = dep_ref[:]
    ihm = ihm_ref[:]
    act_raw_col = act_col_ref[:]
    act_col = (act_raw_col > 0).astype(f32)

    progress = (ts - arr) / (dep - arr)
    w1p = emb_W1_ref[0:1, :]
    w1h = emb_W1_ref[1:2, :]
    x0 = progress * w1p + ihm * w1h + emb_b1_ref[:]
    x = jax.lax.dot_general(x0, emb_W2_ref[:], (((1,), (0,)), ((), ())),
                            preferred_element_type=f32) + emb_b2_ref[:]

    edge_ok = (adj_ref[:] > 0) & (act_row_ref[:] > 0) & (act_raw_col > 0)
    mask_add = jnp.where(edge_ok, 0.0, -1e30).astype(f32)

    h = x
    for l in range(LAYERS):
        W = gat_W_ref[l]
        a_s = gat_a_src_ref[l:l + 1, :]
        a_d = gat_a_dst_ref[l:l + 1, :]
        b = gat_b_ref[l:l + 1, :]

        h1 = jax.lax.dot_general(h, W, (((1,), (0,)), ((), ())),
                                 preferred_element_type=f32)
        asrc_col = jax.lax.dot_general(h1, a_s, (((1,), (1,)), ((), ())),
                                       preferred_element_type=f32)
        adst_col = jax.lax.dot_general(h1, a_d, (((1,), (1,)), ((), ())),
                                       preferred_element_type=f32)
        asrc_row = jax.lax.dot_general(a_s, h1, (((1,), (1,)), ((), ())),
                                       preferred_element_type=f32)
        adst_row = jax.lax.dot_general(a_d, h1, (((1,), (1,)), ((), ())),
                                       preferred_element_type=f32)

        eM = _leaky(asrc_col + adst_row) + mask_add
        e_self_row = _leaky(asrc_row + adst_row)
        m_row = jnp.maximum(jnp.max(eM, axis=0, keepdims=True), e_self_row)
        ex = jnp.exp(eM - m_row)
        denom_row = jnp.sum(ex, axis=0, keepdims=True) \
            + jnp.exp(e_self_row - m_row)

        inv_col = 1.0 / (jnp.transpose(denom_row) + 1e-16)
        m_col = jnp.transpose(m_row)
        e_self_col = _leaky(asrc_col + adst_col)
        ex_self_col = jnp.exp(e_self_col - m_col)

        agg = jax.lax.dot_general(ex, h1, (((0,), (0,)), ((), ())),
                                  preferred_element_type=f32)
        out = agg * inv_col + (ex_self_col * inv_col) * h1 + b
        h = jnp.maximum(out, 0.0) if l < LAYERS - 1 else out

    x = x + h
    mu = jnp.mean(x, axis=1, keepdims=True)
    xc = x - mu
    var = jnp.mean(xc * xc, axis=1, keepdims=True)
    xn = xc * jax.lax.rsqrt(var + 1e-5)
    logit = jax.lax.dot_general(xn, sel_W_ref[:], (((1,), (0,)), ((), ())),
                                preferred_element_type=f32) + sel_b_ref[0, 0]
    y = jax.nn.sigmoid(logit) * act_col
    any_active = jnp.sum(act_raw_col) != 0.0
    out_ref[:] = jnp.where(any_active, y, jnp.zeros_like(y))


def kernel(adjacency_matrix, timestep, arrivals, departures, is_hard_to_match,
           active_agents, emb_W1, emb_b1, emb_W2, emb_b2, gat_W, gat_a_src,
           gat_a_dst, gat_b, sel_W, sel_b):
    f32 = _F32
    ts = jnp.full((1, 1), timestep, f32)
    col = lambda v: v.astype(f32).reshape(N, 1)
    row = lambda v: v.astype(f32).reshape(1, -1)
    args = (
        adjacency_matrix, ts, col(arrivals), col(departures),
        col(is_hard_to_match), col(active_agents), row(active_agents),
        emb_W1, row(emb_b1), emb_W2, row(emb_b2),
        gat_W, gat_a_src, gat_a_dst, gat_b,
        sel_W, row(sel_b),
    )
    out = pl.pallas_call(
        _body,
        out_shape=jax.ShapeDtypeStruct((N, 1), f32),
    )(*args)
    return out


# speedup vs baseline: 6903.6511x; 1.2062x over previous
import jax
import jax.numpy as jnp
from jax.experimental import pallas as pl

N = 1024
HID = 128
LAYERS = 3
_F32 = jnp.float32


def _leaky(x):
    return jnp.where(x >= 0, x, 0.2 * x)


def _body(adj_ref, vecs_ref,
          emb_W1_ref, emb_b1_ref, emb_W2_ref, emb_b2_ref,
          gat_W_ref, gat_a_src_ref, gat_a_dst_ref, gat_b_ref,
          sel_W_ref, sel_b_ref, out_ref):
    f32 = _F32
    arr_row = vecs_ref[0:1, :]
    dep_row = vecs_ref[1:2, :]
    ihm_row = vecs_ref[2:3, :]
    act_raw_row = vecs_ref[3:4, :]
    ts_row = vecs_ref[4:5, :]

    progress_row = (ts_row - arr_row) / (dep_row - arr_row)
    progress = jnp.transpose(progress_row)
    ihm = jnp.transpose(ihm_row)
    act_raw_col = jnp.transpose(act_raw_row)
    act_col = (act_raw_col > 0).astype(f32)

    w1p = emb_W1_ref[0:1, :]
    w1h = emb_W1_ref[1:2, :]
    b1 = emb_b1_ref[:].reshape(1, HID)
    b2 = emb_b2_ref[:].reshape(1, HID)
    x0 = progress * w1p + ihm * w1h + b1
    x = jax.lax.dot_general(x0, emb_W2_ref[:], (((1,), (0,)), ((), ())),
                            preferred_element_type=f32) + b2

    edge_ok = (adj_ref[:] > 0) & (act_raw_row > 0) & (act_raw_col > 0)
    mask_add = jnp.where(edge_ok, 0.0, -1e30).astype(f32)

    h = x
    for l in range(LAYERS):
        W = gat_W_ref[l]
        a_s = gat_a_src_ref[l:l + 1, :]
        a_d = gat_a_dst_ref[l:l + 1, :]
        b = gat_b_ref[l:l + 1, :]

        h1 = jax.lax.dot_general(h, W, (((1,), (0,)), ((), ())),
                                 preferred_element_type=f32)
        asrc_col = jax.lax.dot_general(h1, a_s, (((1,), (1,)), ((), ())),
                                       preferred_element_type=f32)
        adst_col = jax.lax.dot_general(h1, a_d, (((1,), (1,)), ((), ())),
                                       preferred_element_type=f32)
        asrc_row = jax.lax.dot_general(a_s, h1, (((1,), (1,)), ((), ())),
                                       preferred_element_type=f32)
        adst_row = jax.lax.dot_general(a_d, h1, (((1,), (1,)), ((), ())),
                                       preferred_element_type=f32)

        eM = _leaky(asrc_col + adst_row) + mask_add
        e_self_row = _leaky(asrc_row + adst_row)
        m_row = jnp.maximum(jnp.max(eM, axis=0, keepdims=True), e_self_row)
        ex = jnp.exp(eM - m_row)
        denom_row = jnp.sum(ex, axis=0, keepdims=True) \
            + jnp.exp(e_self_row - m_row)

        inv_col = 1.0 / (jnp.transpose(denom_row) + 1e-16)
        m_col = jnp.transpose(m_row)
        e_self_col = _leaky(asrc_col + adst_col)
        ex_self_col = jnp.exp(e_self_col - m_col)

        agg = jax.lax.dot_general(ex, h1, (((0,), (0,)), ((), ())),
                                  preferred_element_type=f32)
        out = agg * inv_col + (ex_self_col * inv_col) * h1 + b
        h = jnp.maximum(out, 0.0) if l < LAYERS - 1 else out

    x = x + h
    mu = jnp.mean(x, axis=1, keepdims=True)
    xc = x - mu
    var = jnp.mean(xc * xc, axis=1, keepdims=True)
    xn = xc * jax.lax.rsqrt(var + 1e-5)
    logit = jax.lax.dot_general(xn, sel_W_ref[:], (((1,), (0,)), ((), ())),
                                preferred_element_type=f32) + sel_b_ref[0]
    y = jax.nn.sigmoid(logit) * act_col
    any_active = jnp.sum(act_raw_col) != 0.0
    out_ref[:] = jnp.where(any_active, y, jnp.zeros_like(y))


def kernel(adjacency_matrix, timestep, arrivals, departures, is_hard_to_match,
           active_agents, emb_W1, emb_b1, emb_W2, emb_b2, gat_W, gat_a_src,
           gat_a_dst, gat_b, sel_W, sel_b):
    f32 = _F32
    vecs = jnp.stack([
        arrivals.astype(f32), departures.astype(f32),
        is_hard_to_match.astype(f32), active_agents.astype(f32),
        jnp.full((N,), timestep, f32),
    ])
    args = (
        adjacency_matrix, vecs,
        emb_W1, emb_b1, emb_W2, emb_b2,
        gat_W, gat_a_src, gat_a_dst, gat_b,
        sel_W, sel_b,
    )
    out = pl.pallas_call(
        _body,
        out_shape=jax.ShapeDtypeStruct((N, 1), f32),
    )(*args)
    return out
